# baseline (speedup 1.0000x reference)
import functools

import jax
import jax.numpy as jnp
from jax.experimental import pallas as pl
from jax.experimental.pallas import tpu as pltpu

N = 10000
DEG = 32
D = 128
DOUT = 128
BN = 200
HALF = N // 2
NBLK = HALF // BN


def _body(self_a, self_b, neigh_a, neigh_b, w_ref, out_ref):
    sa = jnp.sum(neigh_a[...], axis=1) + self_a[...]
    sb = jnp.sum(neigh_b[...], axis=1) + self_b[...]
    m = jnp.concatenate([sa, sb], axis=0) * (1.0 / (DEG + 1))
    r = jnp.maximum(
        jnp.dot(m, w_ref[...], preferred_element_type=jnp.float32), 0.0
    )
    out_ref[...] = r.reshape(2, BN, DOUT)


@jax.jit
def kernel(self_vecs, neigh_vecs, W):
    out = pl.pallas_call(
        _body,
        grid=(NBLK,),
        in_specs=[
            pl.BlockSpec((BN, D), lambda i: (i, 0)),
            pl.BlockSpec((BN, D), lambda i: (i + NBLK, 0)),
            pl.BlockSpec((BN, DEG, D), lambda i: (i, 0, 0)),
            pl.BlockSpec((BN, DEG, D), lambda i: (i + NBLK, 0, 0)),
            pl.BlockSpec((D, DOUT), lambda i: (0, 0)),
        ],
        out_specs=pl.BlockSpec((2, BN, DOUT), lambda i: (0, i, 0)),
        out_shape=jax.ShapeDtypeStruct((2, HALF, DOUT), jnp.float32),
        compiler_params=pltpu.CompilerParams(
            dimension_semantics=("parallel",),
        ),
    )(self_vecs, self_vecs, neigh_vecs, neigh_vecs, W)
    return out.reshape(N, DOUT)
